# baseline (speedup 1.0000x reference)
import functools
import math

import jax
import jax.numpy as jnp
from jax.experimental import pallas as pl
from jax.experimental.pallas import tpu as pltpu

NUM_BREEDS = 120
NUM_EMOTIONS = 8
NUM_ACTIONS = 16
LAMBDA_OBJ = 1.0
LAMBDA_BOX = 5.0
LAMBDA_HEAD = 2.0
LAMBDA_ATTR = 1.0
IGNORE_INDEX = -100

_HI = jax.lax.Precision.HIGHEST


def _smooth_l1(p, t):
    d = jnp.abs(p - t)
    return jnp.where(d < 1.0, 0.5 * d * d, d - 0.5)


def _outer(ones_col, col):
    return jax.lax.dot_general(ones_col, col, (((1,), (1,)), ((), ())),
                               preferred_element_type=jnp.float32,
                               precision=_HI)


def _loss_kernel(body_ref, head_ref, lab_ref, emo_ref, act_ref, hv_ref,
                 img_ref, pred_ref, out_ref, *, B, H, W, C, N):
    R = B * N
    img = img_ref[0, 0]
    sx = img / float(W)
    sy = img / float(H)

    body = body_ref[...]
    head = head_ref[...]
    lab = lab_ref[...]
    emo = emo_ref[...]
    act = act_ref[...]
    hv = hv_ref[...]

    bx1r, by1r = body[:, 0:1], body[:, 1:2]
    bx2r, by2r = body[:, 2:3], body[:, 3:4]
    scale_b = jnp.max(body, axis=1, keepdims=True) <= 1.5
    x1 = jnp.where(scale_b, bx1r * img, bx1r)
    y1 = jnp.where(scale_b, by1r * img, by1r)
    x2 = jnp.where(scale_b, bx2r * img, bx2r)
    y2 = jnp.where(scale_b, by2r * img, by2r)
    bw = x2 - x1
    bh = y2 - y1
    size_ok = (bw > 0) & (bh > 0)
    cx = (x1 + x2) * 0.5
    cy = (y1 + y2) * 0.5
    fgx = cx / sx
    fgy = cy / sy
    gx = fgx.astype(jnp.int32)
    gy = fgy.astype(jnp.int32)
    inb = (gx >= 0) & (gy >= 0) & (gx < W) & (gy < H)
    gxs = jnp.clip(gx, 0, W - 1)
    gys = jnp.clip(gy, 0, H - 1)
    valid = size_ok & inb

    area = (jnp.clip(bx2r - bx1r, 0.0, None) *
            jnp.clip(by2r - by1r, 0.0, None))

    tx = fgx - gx.astype(jnp.float32)
    ty = fgy - gy.astype(jnp.float32)
    safe_bw = jnp.where(bw > 0, bw, 1.0)
    safe_bh = jnp.where(bh > 0, bh, 1.0)
    tw = jnp.log(safe_bw / sx + 1e-06)
    th = jnp.log(safe_bh / sy + 1e-06)

    hx1r, hy1r = head[:, 0:1], head[:, 1:2]
    hx2r, hy2r = head[:, 2:3], head[:, 3:4]
    scale_h = jnp.max(head, axis=1, keepdims=True) <= 1.5
    hx1 = jnp.where(scale_h, hx1r * img, hx1r)
    hy1 = jnp.where(scale_h, hy1r * img, hy1r)
    hx2 = jnp.where(scale_h, hx2r * img, hx2r)
    hy2 = jnp.where(scale_h, hy2r * img, hy2r)
    head_ok = (hv > 0) & ((hx2 - hx1) > 0) & ((hy2 - hy1) > 0)
    rel0 = jnp.clip((hx1 - x1) / safe_bw, 0.0, 1.0)
    rel1 = jnp.clip((hy1 - y1) / safe_bh, 0.0, 1.0)
    rel2 = jnp.clip((hx2 - x1) / safe_bw, 0.0, 1.0)
    rel3 = jnp.clip((hy2 - y1) / safe_bh, 0.0, 1.0)

    rIdx = jax.lax.broadcasted_iota(jnp.int32, (R, R), 0)
    sIdx = jax.lax.broadcasted_iota(jnp.int32, (R, R), 1)
    same_img = (rIdx // N) == (sIdx // N)
    ones_col = jnp.ones((R, 1), dtype=jnp.float32)
    a_row = _outer(ones_col, area)
    before = same_img & ((a_row < area) | ((a_row == area) & (sIdx < rIdx)))
    rank = jnp.sum(before.astype(jnp.float32), axis=1, keepdims=True)

    b_idx = jax.lax.broadcasted_iota(jnp.int32, (R, 1), 0) // N
    cell = b_idx * (H * W) + gys * W + gxs
    cell_f = cell.astype(jnp.float32)
    cell_row = _outer(ones_col, cell_f)
    eqcell = same_img & (cell_row == cell_f) & (sIdx != rIdx)
    eqcell_f = eqcell.astype(jnp.float32)

    pos = jnp.zeros((R, 1), dtype=jnp.float32)
    valid_f = valid.astype(jnp.float32)
    for k in range(N):
        pos_row = _outer(ones_col, pos)
        occ = jnp.sum(eqcell_f * pos_row, axis=1, keepdims=True)
        sel = (rank == float(k))
        newpos = jnp.where(sel & (occ < 0.5), valid_f, 0.0)
        pos = pos + newpos
    total_pos = jnp.sum(pos)

    PAIR = 2 * H * W
    p_idx = jax.lax.broadcasted_iota(jnp.int32, (R, 1), 0) // (2 * N)
    cellp = cell - PAIR * p_idx
    iota_pair = jax.lax.broadcasted_iota(jnp.int32, (R, PAIR), 1)
    onehot = (iota_pair == cellp).astype(jnp.float32)
    gs = []
    for p in range(R // 8):
        gs.append(jax.lax.dot_general(
            onehot[8 * p:8 * p + 8, :], pred_ref[PAIR * p:PAIR * (p + 1), :],
            (((1,), (0,)), ((), ())),
            preferred_element_type=jnp.float32,
            precision=jax.lax.Precision.DEFAULT))
    g = jnp.concatenate(gs, axis=0)

    obj_g = g[:, 0:1]
    braw = g[:, 1:5]
    hraw = g[:, 5:9]
    off = 9
    breed_l = g[:, off:off + NUM_BREEDS]
    off += NUM_BREEDS
    emo_l = g[:, off:off + NUM_EMOTIONS]
    off += NUM_EMOTIONS
    act_l = g[:, off:off + NUM_ACTIONS]

    pxy = 1.0 / (1.0 + jnp.exp(-braw[:, 0:2]))
    txy = jnp.concatenate([tx, ty], axis=1)
    twh = jnp.concatenate([tw, th], axis=1)
    box_r = (jnp.sum(_smooth_l1(pxy, txy), axis=1, keepdims=True) +
             jnp.sum(_smooth_l1(braw[:, 2:4], twh), axis=1, keepdims=True))
    total = LAMBDA_BOX * jnp.sum(pos * box_r)

    ph = 1.0 / (1.0 + jnp.exp(-hraw))
    relm = jnp.concatenate([rel0, rel1, rel2, rel3], axis=1)
    head_r = jnp.sum(_smooth_l1(ph, relm), axis=1, keepdims=True)
    total = total + LAMBDA_HEAD * jnp.sum(pos * head_ok.astype(jnp.float32) * head_r)

    def ce(logits, t, nclass):
        m = jnp.max(logits, axis=1, keepdims=True)
        lse = m + jnp.log(jnp.sum(jnp.exp(logits - m), axis=1, keepdims=True))
        cls_iota = jax.lax.broadcasted_iota(jnp.int32, (R, nclass), 1)
        picked = jnp.sum(jnp.where(cls_iota == t, logits, 0.0), axis=1,
                         keepdims=True)
        return jnp.where(t != IGNORE_INDEX, lse - picked, 0.0)

    attr_r = (ce(breed_l, lab, NUM_BREEDS) +
              ce(emo_l, emo, NUM_EMOTIONS) +
              ce(act_l, act, NUM_ACTIONS))
    total = total + LAMBDA_ATTR * jnp.sum(pos * attr_r)

    o = pred_ref[:, 0:1].reshape(B * H * W // 128, 128)
    base = jnp.sum(jnp.maximum(o, 0.0) + jnp.log(1.0 + jnp.exp(-jnp.abs(o))))
    corr = -jnp.sum(pos * obj_g)
    total = total + LAMBDA_OBJ * (base + corr)

    out_ref[0, 0] = total / jnp.maximum(total_pos, 1.0)


def kernel(pred, body_boxes, head_boxes, labels, emotions, actions,
           head_valid, img_size):
    B, H, W, C = pred.shape
    N = body_boxes.shape[1]
    R = B * N
    pred2d = pred.reshape(B * H * W, C)
    body32 = body_boxes.reshape(R, 4).astype(jnp.float32)
    head32 = head_boxes.reshape(R, 4).astype(jnp.float32)
    lab32 = labels.reshape(R, 1)
    emo32 = emotions.reshape(R, 1)
    act32 = actions.reshape(R, 1)
    hv32 = head_valid.reshape(R, 1).astype(jnp.int32)
    img = jnp.asarray(img_size, jnp.float32).reshape(1, 1)

    out = pl.pallas_call(
        functools.partial(_loss_kernel, B=B, H=H, W=W, C=C, N=N),
        out_shape=jax.ShapeDtypeStruct((1, 1), jnp.float32),
        in_specs=[
            pl.BlockSpec(memory_space=pltpu.VMEM),
            pl.BlockSpec(memory_space=pltpu.VMEM),
            pl.BlockSpec(memory_space=pltpu.VMEM),
            pl.BlockSpec(memory_space=pltpu.VMEM),
            pl.BlockSpec(memory_space=pltpu.VMEM),
            pl.BlockSpec(memory_space=pltpu.VMEM),
            pl.BlockSpec(memory_space=pltpu.SMEM),
            pl.BlockSpec(memory_space=pltpu.VMEM),
        ],
        out_specs=pl.BlockSpec(memory_space=pltpu.SMEM),
    )(body32, head32, lab32, emo32, act32, hv32, img, pred2d)
    return out.reshape(())


# speedup vs baseline: 1.0608x; 1.0608x over previous
import functools
import math

import jax
import jax.numpy as jnp
from jax.experimental import pallas as pl
from jax.experimental.pallas import tpu as pltpu

NUM_BREEDS = 120
NUM_EMOTIONS = 8
NUM_ACTIONS = 16
LAMBDA_OBJ = 1.0
LAMBDA_BOX = 5.0
LAMBDA_HEAD = 2.0
LAMBDA_ATTR = 1.0
IGNORE_INDEX = -100

_HI = jax.lax.Precision.HIGHEST


def _smooth_l1(p, t):
    d = jnp.abs(p - t)
    return jnp.where(d < 1.0, 0.5 * d * d, d - 0.5)


def _outer(ones_col, col):
    return jax.lax.dot_general(ones_col, col, (((1,), (1,)), ((), ())),
                               preferred_element_type=jnp.float32,
                               precision=_HI)


def _loss_kernel(body_ref, head_ref, attr_ref, img_ref, pred_ref, out_ref,
                 g_acc, base_acc, *, B, H, W, C, N, NP):
    R = B * N
    PAIR = B * H * W // NP
    RP = R // NP
    p = pl.program_id(0)
    img = img_ref[0, 0]
    sx = img / float(W)
    sy = img / float(H)

    body = body_ref[...]

    bx1r, by1r = body[:, 0:1], body[:, 1:2]
    bx2r, by2r = body[:, 2:3], body[:, 3:4]
    scale_b = jnp.max(body, axis=1, keepdims=True) <= 1.5
    x1 = jnp.where(scale_b, bx1r * img, bx1r)
    y1 = jnp.where(scale_b, by1r * img, by1r)
    x2 = jnp.where(scale_b, bx2r * img, bx2r)
    y2 = jnp.where(scale_b, by2r * img, by2r)
    bw = x2 - x1
    bh = y2 - y1
    size_ok = (bw > 0) & (bh > 0)
    cx = (x1 + x2) * 0.5
    cy = (y1 + y2) * 0.5
    fgx = cx / sx
    fgy = cy / sy
    gx = fgx.astype(jnp.int32)
    gy = fgy.astype(jnp.int32)
    inb = (gx >= 0) & (gy >= 0) & (gx < W) & (gy < H)
    gxs = jnp.clip(gx, 0, W - 1)
    gys = jnp.clip(gy, 0, H - 1)
    valid = size_ok & inb

    b_idx = jax.lax.broadcasted_iota(jnp.int32, (R, 1), 0) // N
    cell = b_idx * (H * W) + gys * W + gxs

    body8 = body_ref[pl.ds(RP * p, RP), :]
    s8 = jnp.max(body8, axis=1, keepdims=True) <= 1.5
    px1 = jnp.where(s8, body8[:, 0:1] * img, body8[:, 0:1])
    py1 = jnp.where(s8, body8[:, 1:2] * img, body8[:, 1:2])
    px2 = jnp.where(s8, body8[:, 2:3] * img, body8[:, 2:3])
    py2 = jnp.where(s8, body8[:, 3:4] * img, body8[:, 3:4])
    pgx = jnp.clip((((px1 + px2) * 0.5) / sx).astype(jnp.int32), 0, W - 1)
    pgy = jnp.clip((((py1 + py2) * 0.5) / sy).astype(jnp.int32), 0, H - 1)
    b8 = (RP * p + jax.lax.broadcasted_iota(jnp.int32, (RP, 1), 0)) // N
    cell_p = (b8 * (H * W) + pgy * W + pgx) - PAIR * p
    iota_pair = jax.lax.broadcasted_iota(jnp.int32, (RP, PAIR), 1)
    onehot = (iota_pair == cell_p).astype(jnp.float32)
    g_p = jax.lax.dot_general(
        onehot, pred_ref[...], (((1,), (0,)), ((), ())),
        preferred_element_type=jnp.float32,
        precision=jax.lax.Precision.DEFAULT)
    g_acc[pl.ds(RP * p, RP), 0:C] = g_p

    o = pred_ref[:, 0:1].reshape(PAIR // 128, 128)
    part = jnp.sum(jnp.maximum(o, 0.0) + jnp.log(1.0 + jnp.exp(-jnp.abs(o))))

    @pl.when(p == 0)
    def _():
        base_acc[0, 0] = part

    @pl.when(p > 0)
    def _():
        base_acc[0, 0] = base_acc[0, 0] + part

    @pl.when(p == NP - 1)
    def _():
        head = head_ref[...]
        attr = attr_ref[...]
        lab = attr[:, 0:1]
        emo = attr[:, 1:2]
        act = attr[:, 2:3]
        hv = attr[:, 3:4]

        area = (jnp.clip(bx2r - bx1r, 0.0, None) *
                jnp.clip(by2r - by1r, 0.0, None))

        tx = fgx - gx.astype(jnp.float32)
        ty = fgy - gy.astype(jnp.float32)
        safe_bw = jnp.where(bw > 0, bw, 1.0)
        safe_bh = jnp.where(bh > 0, bh, 1.0)
        tw = jnp.log(safe_bw / sx + 1e-06)
        th = jnp.log(safe_bh / sy + 1e-06)

        hx1r, hy1r = head[:, 0:1], head[:, 1:2]
        hx2r, hy2r = head[:, 2:3], head[:, 3:4]
        scale_h = jnp.max(head, axis=1, keepdims=True) <= 1.5
        hx1 = jnp.where(scale_h, hx1r * img, hx1r)
        hy1 = jnp.where(scale_h, hy1r * img, hy1r)
        hx2 = jnp.where(scale_h, hx2r * img, hx2r)
        hy2 = jnp.where(scale_h, hy2r * img, hy2r)
        head_ok = (hv > 0) & ((hx2 - hx1) > 0) & ((hy2 - hy1) > 0)
        rel0 = jnp.clip((hx1 - x1) / safe_bw, 0.0, 1.0)
        rel1 = jnp.clip((hy1 - y1) / safe_bh, 0.0, 1.0)
        rel2 = jnp.clip((hx2 - x1) / safe_bw, 0.0, 1.0)
        rel3 = jnp.clip((hy2 - y1) / safe_bh, 0.0, 1.0)

        rIdx = jax.lax.broadcasted_iota(jnp.int32, (R, R), 0)
        sIdx = jax.lax.broadcasted_iota(jnp.int32, (R, R), 1)
        same_img = (rIdx // N) == (sIdx // N)
        ones_col = jnp.ones((R, 1), dtype=jnp.float32)
        a_row = _outer(ones_col, area)
        before = same_img & ((a_row < area) |
                             ((a_row == area) & (sIdx < rIdx)))
        rank = jnp.sum(before.astype(jnp.float32), axis=1, keepdims=True)

        cell_f = cell.astype(jnp.float32)
        cell_row = _outer(ones_col, cell_f)
        eqcell = same_img & (cell_row == cell_f) & (sIdx != rIdx)
        eqcell_f = eqcell.astype(jnp.float32)

        pos = jnp.zeros((R, 1), dtype=jnp.float32)
        valid_f = valid.astype(jnp.float32)
        for k in range(N):
            pos_row = _outer(ones_col, pos)
            occ = jnp.sum(eqcell_f * pos_row, axis=1, keepdims=True)
            sel = (rank == float(k))
            newpos = jnp.where(sel & (occ < 0.5), valid_f, 0.0)
            pos = pos + newpos
        total_pos = jnp.sum(pos)

        g = g_acc[:, 0:C]
        obj_g = g[:, 0:1]
        braw = g[:, 1:5]
        hraw = g[:, 5:9]
        off = 9
        breed_l = g[:, off:off + NUM_BREEDS]
        off += NUM_BREEDS
        emo_l = g[:, off:off + NUM_EMOTIONS]
        off += NUM_EMOTIONS
        act_l = g[:, off:off + NUM_ACTIONS]

        pxy = 1.0 / (1.0 + jnp.exp(-braw[:, 0:2]))
        txy = jnp.concatenate([tx, ty], axis=1)
        twh = jnp.concatenate([tw, th], axis=1)
        box_r = (jnp.sum(_smooth_l1(pxy, txy), axis=1, keepdims=True) +
                 jnp.sum(_smooth_l1(braw[:, 2:4], twh), axis=1,
                         keepdims=True))
        total = LAMBDA_BOX * jnp.sum(pos * box_r)

        ph = 1.0 / (1.0 + jnp.exp(-hraw))
        relm = jnp.concatenate([rel0, rel1, rel2, rel3], axis=1)
        head_r = jnp.sum(_smooth_l1(ph, relm), axis=1, keepdims=True)
        total = total + LAMBDA_HEAD * jnp.sum(
            pos * head_ok.astype(jnp.float32) * head_r)

        def ce(logits, t, nclass):
            m = jnp.max(logits, axis=1, keepdims=True)
            lse = m + jnp.log(jnp.sum(jnp.exp(logits - m), axis=1,
                                      keepdims=True))
            cls_iota = jax.lax.broadcasted_iota(jnp.int32, (R, nclass), 1)
            picked = jnp.sum(jnp.where(cls_iota == t, logits, 0.0), axis=1,
                             keepdims=True)
            return jnp.where(t != IGNORE_INDEX, lse - picked, 0.0)

        attr_r = (ce(breed_l, lab, NUM_BREEDS) +
                  ce(emo_l, emo, NUM_EMOTIONS) +
                  ce(act_l, act, NUM_ACTIONS))
        total = total + LAMBDA_ATTR * jnp.sum(pos * attr_r)

        corr = -jnp.sum(pos * obj_g)
        total = total + LAMBDA_OBJ * (base_acc[0, 0] + corr)

        out_ref[0, 0] = total / jnp.maximum(total_pos, 1.0)


def kernel(pred, body_boxes, head_boxes, labels, emotions, actions,
           head_valid, img_size):
    B, H, W, C = pred.shape
    N = body_boxes.shape[1]
    R = B * N
    NP = 4
    PAIR = B * H * W // NP
    pred2d = pred.reshape(B * H * W, C)
    body32 = body_boxes.reshape(R, 4).astype(jnp.float32)
    head32 = head_boxes.reshape(R, 4).astype(jnp.float32)
    attr32 = jnp.stack([labels.reshape(R), emotions.reshape(R),
                        actions.reshape(R),
                        head_valid.reshape(R).astype(jnp.int32)],
                       axis=-1).astype(jnp.int32)
    img = jnp.asarray(img_size, jnp.float32).reshape(1, 1)

    out = pl.pallas_call(
        functools.partial(_loss_kernel, B=B, H=H, W=W, C=C, N=N, NP=NP),
        grid=(NP,),
        out_shape=jax.ShapeDtypeStruct((1, 1), jnp.float32),
        in_specs=[
            pl.BlockSpec((R, 4), lambda p: (0, 0)),
            pl.BlockSpec((R, 4), lambda p: (0, 0)),
            pl.BlockSpec((R, 4), lambda p: (0, 0)),
            pl.BlockSpec(memory_space=pltpu.SMEM),
            pl.BlockSpec((PAIR, C), lambda p: (p, 0)),
        ],
        out_specs=pl.BlockSpec(memory_space=pltpu.SMEM),
        scratch_shapes=[pltpu.VMEM((R, 160), jnp.float32),
                        pltpu.SMEM((1, 1), jnp.float32)],
    )(body32, head32, attr32, img, pred2d)
    return out.reshape(())


# speedup vs baseline: 1.0895x; 1.0271x over previous
import functools
import math

import jax
import jax.numpy as jnp
from jax.experimental import pallas as pl
from jax.experimental.pallas import tpu as pltpu

NUM_BREEDS = 120
NUM_EMOTIONS = 8
NUM_ACTIONS = 16
LAMBDA_OBJ = 1.0
LAMBDA_BOX = 5.0
LAMBDA_HEAD = 2.0
LAMBDA_ATTR = 1.0
IGNORE_INDEX = -100

_HI = jax.lax.Precision.HIGHEST


def _smooth_l1(p, t):
    d = jnp.abs(p - t)
    return jnp.where(d < 1.0, 0.5 * d * d, d - 0.5)


def _outer(ones_col, col):
    return jax.lax.dot_general(ones_col, col, (((1,), (1,)), ((), ())),
                               preferred_element_type=jnp.float32,
                               precision=_HI)


def _loss_kernel(body_ref, head_ref, attr_ref, img_ref, pred_ref, out_ref,
                 g_acc, base_acc, *, B, H, W, C, N, NP):
    R = B * N
    PAIR = B * H * W // NP
    RP = R // NP
    p = pl.program_id(0)
    img = img_ref[0, 0]
    sx = img / float(W)
    sy = img / float(H)

    body = body_ref[...]

    bx1r, by1r = body[:, 0:1], body[:, 1:2]
    bx2r, by2r = body[:, 2:3], body[:, 3:4]
    scale_b = jnp.max(body, axis=1, keepdims=True) <= 1.5
    x1 = jnp.where(scale_b, bx1r * img, bx1r)
    y1 = jnp.where(scale_b, by1r * img, by1r)
    x2 = jnp.where(scale_b, bx2r * img, bx2r)
    y2 = jnp.where(scale_b, by2r * img, by2r)
    bw = x2 - x1
    bh = y2 - y1
    size_ok = (bw > 0) & (bh > 0)
    cx = (x1 + x2) * 0.5
    cy = (y1 + y2) * 0.5
    fgx = cx / sx
    fgy = cy / sy
    gx = fgx.astype(jnp.int32)
    gy = fgy.astype(jnp.int32)
    inb = (gx >= 0) & (gy >= 0) & (gx < W) & (gy < H)
    gxs = jnp.clip(gx, 0, W - 1)
    gys = jnp.clip(gy, 0, H - 1)
    valid = size_ok & inb

    b_idx = jax.lax.broadcasted_iota(jnp.int32, (R, 1), 0) // N
    cell = b_idx * (H * W) + gys * W + gxs

    body8 = body_ref[pl.ds(RP * p, RP), :]
    s8 = jnp.max(body8, axis=1, keepdims=True) <= 1.5
    px1 = jnp.where(s8, body8[:, 0:1] * img, body8[:, 0:1])
    py1 = jnp.where(s8, body8[:, 1:2] * img, body8[:, 1:2])
    px2 = jnp.where(s8, body8[:, 2:3] * img, body8[:, 2:3])
    py2 = jnp.where(s8, body8[:, 3:4] * img, body8[:, 3:4])
    pgx = jnp.clip((((px1 + px2) * 0.5) / sx).astype(jnp.int32), 0, W - 1)
    pgy = jnp.clip((((py1 + py2) * 0.5) / sy).astype(jnp.int32), 0, H - 1)
    b8 = (RP * p + jax.lax.broadcasted_iota(jnp.int32, (RP, 1), 0)) // N
    cell_p = (b8 * (H * W) + pgy * W + pgx) - PAIR * p
    iota_pair = jax.lax.broadcasted_iota(jnp.int32, (RP, PAIR), 1)
    onehot = (iota_pair == cell_p).astype(jnp.float32)
    g_p = jax.lax.dot_general(
        onehot, pred_ref[...], (((1,), (0,)), ((), ())),
        preferred_element_type=jnp.float32,
        precision=jax.lax.Precision.DEFAULT)
    g_acc[pl.ds(RP * p, RP), 0:C] = g_p

    o = pred_ref[:, 0:1].reshape(PAIR // 128, 128)
    part = jnp.sum(jnp.maximum(o, 0.0) + jnp.log(1.0 + jnp.exp(-jnp.abs(o))))

    @pl.when(p == 0)
    def _():
        base_acc[0, 0] = part

    @pl.when(p > 0)
    def _():
        base_acc[0, 0] = base_acc[0, 0] + part

    @pl.when(p == NP - 1)
    def _():
        head = head_ref[...]
        attr = attr_ref[...]
        lab = attr[:, 0:1]
        emo = attr[:, 1:2]
        act = attr[:, 2:3]
        hv = attr[:, 3:4]

        area = (jnp.clip(bx2r - bx1r, 0.0, None) *
                jnp.clip(by2r - by1r, 0.0, None))

        tx = fgx - gx.astype(jnp.float32)
        ty = fgy - gy.astype(jnp.float32)
        safe_bw = jnp.where(bw > 0, bw, 1.0)
        safe_bh = jnp.where(bh > 0, bh, 1.0)
        tw = jnp.log(safe_bw / sx + 1e-06)
        th = jnp.log(safe_bh / sy + 1e-06)

        hx1r, hy1r = head[:, 0:1], head[:, 1:2]
        hx2r, hy2r = head[:, 2:3], head[:, 3:4]
        scale_h = jnp.max(head, axis=1, keepdims=True) <= 1.5
        hx1 = jnp.where(scale_h, hx1r * img, hx1r)
        hy1 = jnp.where(scale_h, hy1r * img, hy1r)
        hx2 = jnp.where(scale_h, hx2r * img, hx2r)
        hy2 = jnp.where(scale_h, hy2r * img, hy2r)
        head_ok = (hv > 0) & ((hx2 - hx1) > 0) & ((hy2 - hy1) > 0)
        rel0 = jnp.clip((hx1 - x1) / safe_bw, 0.0, 1.0)
        rel1 = jnp.clip((hy1 - y1) / safe_bh, 0.0, 1.0)
        rel2 = jnp.clip((hx2 - x1) / safe_bw, 0.0, 1.0)
        rel3 = jnp.clip((hy2 - y1) / safe_bh, 0.0, 1.0)

        rIdx = jax.lax.broadcasted_iota(jnp.int32, (R, R), 0)
        sIdx = jax.lax.broadcasted_iota(jnp.int32, (R, R), 1)
        same_img = (rIdx // N) == (sIdx // N)
        ones_col = jnp.ones((R, 1), dtype=jnp.float32)
        a_row = _outer(ones_col, area)
        before = same_img & ((a_row < area) |
                             ((a_row == area) & (sIdx < rIdx)))
        rank = jnp.sum(before.astype(jnp.float32), axis=1, keepdims=True)

        cell_f = cell.astype(jnp.float32)
        cell_row = _outer(ones_col, cell_f)
        eqcell = same_img & (cell_row == cell_f) & (sIdx != rIdx)
        eqcell_f = eqcell.astype(jnp.float32)

        pos = jnp.zeros((R, 1), dtype=jnp.float32)
        valid_f = valid.astype(jnp.float32)
        for k in range(N):
            pos_row = _outer(ones_col, pos)
            occ = jnp.sum(eqcell_f * pos_row, axis=1, keepdims=True)
            sel = (rank == float(k))
            newpos = jnp.where(sel & (occ < 0.5), valid_f, 0.0)
            pos = pos + newpos
        total_pos = jnp.sum(pos)

        g = g_acc[:, 0:C]
        obj_g = g[:, 0:1]
        braw = g[:, 1:5]
        hraw = g[:, 5:9]
        off = 9
        breed_l = g[:, off:off + NUM_BREEDS]
        off += NUM_BREEDS
        emo_l = g[:, off:off + NUM_EMOTIONS]
        off += NUM_EMOTIONS
        act_l = g[:, off:off + NUM_ACTIONS]

        pxy = 1.0 / (1.0 + jnp.exp(-braw[:, 0:2]))
        txy = jnp.concatenate([tx, ty], axis=1)
        twh = jnp.concatenate([tw, th], axis=1)
        box_r = (jnp.sum(_smooth_l1(pxy, txy), axis=1, keepdims=True) +
                 jnp.sum(_smooth_l1(braw[:, 2:4], twh), axis=1,
                         keepdims=True))
        total = LAMBDA_BOX * jnp.sum(pos * box_r)

        ph = 1.0 / (1.0 + jnp.exp(-hraw))
        relm = jnp.concatenate([rel0, rel1, rel2, rel3], axis=1)
        head_r = jnp.sum(_smooth_l1(ph, relm), axis=1, keepdims=True)
        total = total + LAMBDA_HEAD * jnp.sum(
            pos * head_ok.astype(jnp.float32) * head_r)

        def ce(logits, t, nclass):
            m = jnp.max(logits, axis=1, keepdims=True)
            lse = m + jnp.log(jnp.sum(jnp.exp(logits - m), axis=1,
                                      keepdims=True))
            cls_iota = jax.lax.broadcasted_iota(jnp.int32, (R, nclass), 1)
            picked = jnp.sum(jnp.where(cls_iota == t, logits, 0.0), axis=1,
                             keepdims=True)
            return jnp.where(t != IGNORE_INDEX, lse - picked, 0.0)

        attr_r = (ce(breed_l, lab, NUM_BREEDS) +
                  ce(emo_l, emo, NUM_EMOTIONS) +
                  ce(act_l, act, NUM_ACTIONS))
        total = total + LAMBDA_ATTR * jnp.sum(pos * attr_r)

        corr = -jnp.sum(pos * obj_g)
        total = total + LAMBDA_OBJ * (base_acc[0, 0] + corr)

        out_ref[0, 0] = total / jnp.maximum(total_pos, 1.0)


def kernel(pred, body_boxes, head_boxes, labels, emotions, actions,
           head_valid, img_size):
    B, H, W, C = pred.shape
    N = body_boxes.shape[1]
    R = B * N
    NP = 2
    PAIR = B * H * W // NP
    pred2d = pred.reshape(B * H * W, C)
    body32 = body_boxes.reshape(R, 4).astype(jnp.float32)
    head32 = head_boxes.reshape(R, 4).astype(jnp.float32)
    attr32 = jnp.stack([labels.reshape(R), emotions.reshape(R),
                        actions.reshape(R),
                        head_valid.reshape(R).astype(jnp.int32)],
                       axis=-1).astype(jnp.int32)
    img = jnp.asarray(img_size, jnp.float32).reshape(1, 1)

    out = pl.pallas_call(
        functools.partial(_loss_kernel, B=B, H=H, W=W, C=C, N=N, NP=NP),
        grid=(NP,),
        out_shape=jax.ShapeDtypeStruct((1, 1), jnp.float32),
        in_specs=[
            pl.BlockSpec((R, 4), lambda p: (0, 0)),
            pl.BlockSpec((R, 4), lambda p: (0, 0)),
            pl.BlockSpec((R, 4), lambda p: (0, 0)),
            pl.BlockSpec(memory_space=pltpu.SMEM),
            pl.BlockSpec((PAIR, C), lambda p: (p, 0)),
        ],
        out_specs=pl.BlockSpec(memory_space=pltpu.SMEM),
        scratch_shapes=[pltpu.VMEM((R, 160), jnp.float32),
                        pltpu.SMEM((1, 1), jnp.float32)],
    )(body32, head32, attr32, img, pred2d)
    return out.reshape(())


# speedup vs baseline: 1.0926x; 1.0028x over previous
import functools

import jax
import jax.numpy as jnp
from jax.experimental import pallas as pl
from jax.experimental.pallas import tpu as pltpu

NUM_BREEDS = 120
NUM_EMOTIONS = 8
NUM_ACTIONS = 16
LAMBDA_OBJ = 1.0
LAMBDA_BOX = 5.0
LAMBDA_HEAD = 2.0
LAMBDA_ATTR = 1.0
IGNORE_INDEX = -100

_HI = jax.lax.Precision.HIGHEST


def _smooth_l1(p, t):
    d = jnp.abs(p - t)
    return jnp.where(d < 1.0, 0.5 * d * d, d - 0.5)


def _outer(ones_col, col):
    return jax.lax.dot_general(ones_col, col, (((1,), (1,)), ((), ())),
                               preferred_element_type=jnp.float32,
                               precision=_HI)


def _loss_kernel(body_ref, head_ref, attr_ref, img_ref, pred_ref, out_ref,
                 g_acc, base_acc, *, B, H, W, C, N, NP):
    R = B * N
    PAIR = B * H * W // NP
    RP = R // NP
    p = pl.program_id(0)
    img = img_ref[0, 0]
    sx = img / float(W)
    sy = img / float(H)

    body = body_ref[...]

    bx1r, by1r = body[:, 0:1], body[:, 1:2]
    bx2r, by2r = body[:, 2:3], body[:, 3:4]
    scale_b = jnp.max(body, axis=1, keepdims=True) <= 1.5
    x1 = jnp.where(scale_b, bx1r * img, bx1r)
    y1 = jnp.where(scale_b, by1r * img, by1r)
    x2 = jnp.where(scale_b, bx2r * img, bx2r)
    y2 = jnp.where(scale_b, by2r * img, by2r)
    bw = x2 - x1
    bh = y2 - y1
    size_ok = (bw > 0) & (bh > 0)
    cx = (x1 + x2) * 0.5
    cy = (y1 + y2) * 0.5
    fgx = cx / sx
    fgy = cy / sy
    gx = fgx.astype(jnp.int32)
    gy = fgy.astype(jnp.int32)
    inb = (gx >= 0) & (gy >= 0) & (gx < W) & (gy < H)
    gxs = jnp.clip(gx, 0, W - 1)
    gys = jnp.clip(gy, 0, H - 1)
    valid = size_ok & inb

    b_idx = jax.lax.broadcasted_iota(jnp.int32, (R, 1), 0) // N
    cell = b_idx * (H * W) + gys * W + gxs

    body8 = body_ref[pl.ds(RP * p, RP), :]
    s8 = jnp.max(body8, axis=1, keepdims=True) <= 1.5
    px1 = jnp.where(s8, body8[:, 0:1] * img, body8[:, 0:1])
    py1 = jnp.where(s8, body8[:, 1:2] * img, body8[:, 1:2])
    px2 = jnp.where(s8, body8[:, 2:3] * img, body8[:, 2:3])
    py2 = jnp.where(s8, body8[:, 3:4] * img, body8[:, 3:4])
    pgx = jnp.clip((((px1 + px2) * 0.5) / sx).astype(jnp.int32), 0, W - 1)
    pgy = jnp.clip((((py1 + py2) * 0.5) / sy).astype(jnp.int32), 0, H - 1)
    b8 = (RP * p + jax.lax.broadcasted_iota(jnp.int32, (RP, 1), 0)) // N
    cell_p = (b8 * (H * W) + pgy * W + pgx) - PAIR * p
    iota_pair = jax.lax.broadcasted_iota(jnp.int32, (RP, PAIR), 1)
    onehot = (iota_pair == cell_p).astype(jnp.float32)
    g_p = jax.lax.dot_general(
        onehot, pred_ref[...], (((1,), (0,)), ((), ())),
        preferred_element_type=jnp.float32,
        precision=jax.lax.Precision.DEFAULT)
    g_acc[pl.ds(RP * p, RP), 0:C] = g_p

    o = pred_ref[:, 0:1].reshape(PAIR // 128, 128)
    part = jnp.sum(jnp.maximum(o, 0.0) + jnp.log(1.0 + jnp.exp(-jnp.abs(o))))

    @pl.when(p == 0)
    def _():
        base_acc[0, 0] = part

    @pl.when(p > 0)
    def _():
        base_acc[0, 0] = base_acc[0, 0] + part

    @pl.when(p == NP - 1)
    def _():
        head = head_ref[...]
        attr = attr_ref[...]
        lab = attr[:, 0:1]
        emo = attr[:, 1:2]
        act = attr[:, 2:3]
        hv = attr[:, 3:4]

        area = (jnp.clip(bx2r - bx1r, 0.0, None) *
                jnp.clip(by2r - by1r, 0.0, None))

        tx = fgx - gx.astype(jnp.float32)
        ty = fgy - gy.astype(jnp.float32)
        safe_bw = jnp.where(bw > 0, bw, 1.0)
        safe_bh = jnp.where(bh > 0, bh, 1.0)
        tw = jnp.log(safe_bw / sx + 1e-06)
        th = jnp.log(safe_bh / sy + 1e-06)

        hx1r, hy1r = head[:, 0:1], head[:, 1:2]
        hx2r, hy2r = head[:, 2:3], head[:, 3:4]
        scale_h = jnp.max(head, axis=1, keepdims=True) <= 1.5
        hx1 = jnp.where(scale_h, hx1r * img, hx1r)
        hy1 = jnp.where(scale_h, hy1r * img, hy1r)
        hx2 = jnp.where(scale_h, hx2r * img, hx2r)
        hy2 = jnp.where(scale_h, hy2r * img, hy2r)
        head_ok = (hv > 0) & ((hx2 - hx1) > 0) & ((hy2 - hy1) > 0)
        rel0 = jnp.clip((hx1 - x1) / safe_bw, 0.0, 1.0)
        rel1 = jnp.clip((hy1 - y1) / safe_bh, 0.0, 1.0)
        rel2 = jnp.clip((hx2 - x1) / safe_bw, 0.0, 1.0)
        rel3 = jnp.clip((hy2 - y1) / safe_bh, 0.0, 1.0)

        rIdx = jax.lax.broadcasted_iota(jnp.int32, (R, R), 0)
        sIdx = jax.lax.broadcasted_iota(jnp.int32, (R, R), 1)
        same_img = (rIdx // N) == (sIdx // N)
        ones_col = jnp.ones((R, 1), dtype=jnp.float32)
        a_row = _outer(ones_col, area)
        before = same_img & ((a_row < area) |
                             ((a_row == area) & (sIdx < rIdx)))
        rank = jnp.sum(before.astype(jnp.float32), axis=1, keepdims=True)

        cell_f = cell.astype(jnp.float32)
        cell_row = _outer(ones_col, cell_f)
        eqcell = same_img & (cell_row == cell_f) & (sIdx != rIdx)
        eqcell_f = eqcell.astype(jnp.float32)

        pos = jnp.zeros((R, 1), dtype=jnp.float32)
        valid_f = valid.astype(jnp.float32)
        for k in range(N):
            pos_row = _outer(ones_col, pos)
            occ = jnp.sum(eqcell_f * pos_row, axis=1, keepdims=True)
            sel = (rank == float(k))
            newpos = jnp.where(sel & (occ < 0.5), valid_f, 0.0)
            pos = pos + newpos
        total_pos = jnp.sum(pos)

        g = g_acc[:, 0:C]
        obj_g = g[:, 0:1]
        braw = g[:, 1:5]
        hraw = g[:, 5:9]
        off = 9
        breed_l = g[:, off:off + NUM_BREEDS]
        off += NUM_BREEDS
        emo_l = g[:, off:off + NUM_EMOTIONS]
        off += NUM_EMOTIONS
        act_l = g[:, off:off + NUM_ACTIONS]

        pxy = 1.0 / (1.0 + jnp.exp(-braw[:, 0:2]))
        txy = jnp.concatenate([tx, ty], axis=1)
        twh = jnp.concatenate([tw, th], axis=1)
        box_r = (jnp.sum(_smooth_l1(pxy, txy), axis=1, keepdims=True) +
                 jnp.sum(_smooth_l1(braw[:, 2:4], twh), axis=1,
                         keepdims=True))
        total = LAMBDA_BOX * jnp.sum(pos * box_r)

        ph = 1.0 / (1.0 + jnp.exp(-hraw))
        relm = jnp.concatenate([rel0, rel1, rel2, rel3], axis=1)
        head_r = jnp.sum(_smooth_l1(ph, relm), axis=1, keepdims=True)
        total = total + LAMBDA_HEAD * jnp.sum(
            pos * head_ok.astype(jnp.float32) * head_r)

        def ce(logits, t, nclass):
            m = jnp.max(logits, axis=1, keepdims=True)
            lse = m + jnp.log(jnp.sum(jnp.exp(logits - m), axis=1,
                                      keepdims=True))
            cls_iota = jax.lax.broadcasted_iota(jnp.int32, (R, nclass), 1)
            picked = jnp.sum(jnp.where(cls_iota == t, logits, 0.0), axis=1,
                             keepdims=True)
            return jnp.where(t != IGNORE_INDEX, lse - picked, 0.0)

        attr_r = (ce(breed_l, lab, NUM_BREEDS) +
                  ce(emo_l, emo, NUM_EMOTIONS) +
                  ce(act_l, act, NUM_ACTIONS))
        total = total + LAMBDA_ATTR * jnp.sum(pos * attr_r)

        corr = -jnp.sum(pos * obj_g)
        total = total + LAMBDA_OBJ * (base_acc[0, 0] + corr)

        out_ref[0, 0] = total / jnp.maximum(total_pos, 1.0)


def kernel(pred, body_boxes, head_boxes, labels, emotions, actions,
           head_valid, img_size):
    B, H, W, C = pred.shape
    N = body_boxes.shape[1]
    R = B * N
    NP = 2
    PAIR = B * H * W // NP
    pred2d = pred.reshape(B * H * W, C)
    body32 = body_boxes.reshape(R, 4).astype(jnp.float32)
    head32 = head_boxes.reshape(R, 4).astype(jnp.float32)
    attr32 = jnp.stack([labels.reshape(R), emotions.reshape(R),
                        actions.reshape(R),
                        head_valid.reshape(R).astype(jnp.int32)],
                       axis=-1).astype(jnp.int32)
    img = jnp.asarray(img_size, jnp.float32).reshape(1, 1)

    out = pl.pallas_call(
        functools.partial(_loss_kernel, B=B, H=H, W=W, C=C, N=N, NP=NP),
        grid=(NP,),
        out_shape=jax.ShapeDtypeStruct((1, 1), jnp.float32),
        in_specs=[
            pl.BlockSpec((R, 4), lambda p: (0, 0)),
            pl.BlockSpec((R, 4), lambda p: (0, 0)),
            pl.BlockSpec((R, 4), lambda p: (0, 0)),
            pl.BlockSpec(memory_space=pltpu.SMEM),
            pl.BlockSpec((PAIR, C), lambda p: (p, 0)),
        ],
        out_specs=pl.BlockSpec(memory_space=pltpu.SMEM),
        scratch_shapes=[pltpu.VMEM((R, 160), jnp.float32),
                        pltpu.SMEM((1, 1), jnp.float32)],
    )(body32, head32, attr32, img, pred2d)
    return out.reshape(())
